# Initial kernel scaffold; baseline (speedup 1.0000x reference)
import jax
import jax.numpy as jnp
from jax.experimental import pallas as pl


def kernel(x, edge_index, batch, W_in, b_in, W_root, W_rel, b_rel, W1, b1, W2, b2, W3, b3):
    raise NotImplementedError("write your pallas kernel here")


# speedup vs baseline: 3.0577x; 3.0577x over previous
import functools

import jax
import jax.numpy as jnp
from jax import lax
from jax.experimental import pallas as pl
from jax.experimental.pallas import tpu as pltpu
from jax.experimental.pallas import tpu_sc as plsc

N = 10000
E = 320000
D = 128
G = 256
L = 3

NW = 32
EDGES_PER_W = 10240
E_PAD = NW * EDGES_PER_W
NPAD = 10240
CHUNK = 128
K = 8
ROWS_OUT = N // 16


def _sc_edge_agg():
    mesh = plsc.VectorSubcoreMesh(core_axis_name="c", subcore_axis_name="s",
                                  num_cores=2, num_subcores=16)

    @functools.partial(
        pl.kernel,
        out_type=jax.ShapeDtypeStruct((2 * N, D), jnp.float32),
        mesh=mesh,
        scratch_types=[
            pltpu.VMEM((K, CHUNK), jnp.int32),
            pltpu.VMEM((K, CHUNK), jnp.int32),
            pltpu.VMEM((CHUNK, D), jnp.float32),
            pltpu.VMEM_SHARED((NPAD, D), jnp.float32),
            pltpu.SemaphoreType.DMA,
        ],
    )
    def k(h_hbm, src_hbm, dst_hbm, out_hbm, src_v, dst_v, rows_v, agg_sh, sem):
        cid = lax.axis_index("c")
        sid = lax.axis_index("s")
        wid = cid * 16 + sid

        def zrow(r, carry):
            for c8 in range(D // 16):
                rows_v[r, pl.ds(c8 * 16, 16)] = jnp.zeros((16,), jnp.float32)
            return carry
        lax.fori_loop(0, CHUNK, zrow, 0)
        for z in range(NPAD // 16 // CHUNK):
            pltpu.sync_copy(rows_v, agg_sh.at[pl.ds(sid * (NPAD // 16) + z * CHUNK, CHUNK)])
        plsc.subcore_barrier()

        base_row = wid * (EDGES_PER_W // CHUNK)
        def chunk_body(t, carry):
            pltpu.sync_copy(src_hbm.at[pl.ds(base_row + t * K, K)], src_v)
            pltpu.sync_copy(dst_hbm.at[pl.ds(base_row + t * K, K)], dst_v)
            for j in range(K):
                pltpu.async_copy(h_hbm.at[src_v.at[j]], rows_v, sem).wait()
                pltpu.sync_copy(rows_v, agg_sh.at[dst_v.at[j]], add=True)
            return carry
        lax.fori_loop(0, EDGES_PER_W // CHUNK // K, chunk_body, 0)

        plsc.subcore_barrier()
        @pl.when(sid < 15)
        def _():
            pltpu.sync_copy(
                agg_sh.at[pl.ds(sid * 624, 624)],
                out_hbm.at[pl.ds(cid * N + sid * 624, 624)],
            )

        @pl.when(sid == 15)
        def _():
            pltpu.sync_copy(
                agg_sh.at[pl.ds(15 * 624, 640)],
                out_hbm.at[pl.ds(cid * N + 15 * 624, 640)],
            )

    return k


_sc_cache = {}


def _sc_edge_agg_fn(h, src2d, dst2d):
    if "k" not in _sc_cache:
        _sc_cache["k"] = _sc_edge_agg()
    return _sc_cache["k"](h, src2d, dst2d)


R = 1000
NBLK = N // R


def _tc_in(x_ref, w_ref, b_ref, o_ref):
    o_ref[...] = jnp.maximum(
        jnp.dot(x_ref[...], w_ref[...], preferred_element_type=jnp.float32)
        + b_ref[...], 0.0)


def _tc_layer(p0_ref, p1_ref, h_ref, wrel_ref, wroot_ref, b_ref, o_ref):
    agg = p0_ref[...] + p1_ref[...]
    o_ref[...] = jnp.maximum(
        jnp.dot(agg, wrel_ref[...], preferred_element_type=jnp.float32)
        + jnp.dot(h_ref[...], wroot_ref[...], preferred_element_type=jnp.float32)
        + b_ref[...], 0.0)


def _tc_pool(h_ref, batch_ref, w1_ref, b1_ref, w2_ref, b2_ref, w3_ref, b3_ref,
             o_ref, sums_s, cnt_s):
    i = pl.program_id(0)

    @pl.when(i == 0)
    def _():
        sums_s[...] = jnp.zeros_like(sums_s)
        cnt_s[...] = jnp.zeros_like(cnt_s)

    ids = lax.broadcasted_iota(jnp.int32, (R, G), 1)
    oh = (batch_ref[...] == ids).astype(jnp.float32)
    dn = (((0,), (0,)), ((), ()))
    sums_s[...] += lax.dot_general(oh, h_ref[...], dn,
                                   precision=lax.Precision.HIGHEST,
                                   preferred_element_type=jnp.float32)
    cnt_s[...] += lax.dot_general(oh, jnp.ones((R, D), jnp.float32), dn,
                                  precision=lax.Precision.HIGHEST,
                                  preferred_element_type=jnp.float32)

    @pl.when(i == NBLK - 1)
    def _():
        pooled = sums_s[...] / jnp.maximum(cnt_s[...], 1.0)
        m = jnp.maximum(
            jnp.dot(pooled, w1_ref[...], preferred_element_type=jnp.float32)
            + b1_ref[...], 0.0)
        m = jnp.maximum(
            jnp.dot(m, w2_ref[...], preferred_element_type=jnp.float32)
            + b2_ref[...], 0.0)
        o_ref[...] = (jnp.dot(m, w3_ref[...], preferred_element_type=jnp.float32)
                      + b3_ref[...])


def kernel(x, edge_index, batch, W_in, b_in, W_root, W_rel, b_rel, W1, b1, W2, b2, W3, b3):
    src = edge_index[0]
    dst = edge_index[1]
    pad = E_PAD - E
    src2d = jnp.concatenate([src, jnp.zeros((pad,), jnp.int32)]).reshape(E_PAD // CHUNK, CHUNK)
    dst2d = jnp.concatenate([dst, jnp.full((pad,), N, jnp.int32)]).reshape(E_PAD // CHUNK, CHUNK)

    full = pl.BlockSpec((R, D), lambda i: (i, 0))
    wspec = pl.BlockSpec((D, D), lambda i: (0, 0))
    bspec = pl.BlockSpec((1, D), lambda i: (0, 0))

    h = pl.pallas_call(
        _tc_in,
        grid=(NBLK,),
        in_specs=[full, wspec, bspec],
        out_specs=full,
        out_shape=jax.ShapeDtypeStruct((N, D), jnp.float32),
    )(x, W_in, b_in.reshape(1, D))

    for i in range(L):
        pflat = _sc_edge_agg_fn(h, src2d, dst2d)
        h = pl.pallas_call(
            _tc_layer,
            grid=(NBLK,),
            in_specs=[
                pl.BlockSpec((R, D), lambda i: (i, 0)),
                pl.BlockSpec((R, D), lambda i: (NBLK + i, 0)),
                full, wspec, wspec, bspec,
            ],
            out_specs=full,
            out_shape=jax.ShapeDtypeStruct((N, D), jnp.float32),
        )(pflat, pflat, h, W_rel[i], W_root[i], b_rel[i].reshape(1, D))

    out = pl.pallas_call(
        _tc_pool,
        grid=(NBLK,),
        in_specs=[
            full,
            pl.BlockSpec((R, 1), lambda i: (i, 0)),
            pl.BlockSpec((D, D), lambda i: (0, 0)),
            pl.BlockSpec((1, D), lambda i: (0, 0)),
            pl.BlockSpec((D, D // 2), lambda i: (0, 0)),
            pl.BlockSpec((1, D // 2), lambda i: (0, 0)),
            pl.BlockSpec((D // 2, 1), lambda i: (0, 0)),
            pl.BlockSpec((1, 1), lambda i: (0, 0)),
        ],
        out_specs=pl.BlockSpec((G, 1), lambda i: (0, 0)),
        out_shape=jax.ShapeDtypeStruct((G, 1), jnp.float32),
        scratch_shapes=[
            pltpu.VMEM((G, D), jnp.float32),
            pltpu.VMEM((G, D), jnp.float32),
        ],
    )(h, batch.reshape(N, 1), W1, b1.reshape(1, D), W2, b2.reshape(1, D // 2),
      W3, b3.reshape(1, 1))

    return out
